# baseline (speedup 1.0000x reference)
import functools

import jax
import jax.numpy as jnp
from jax import lax
from jax.experimental import pallas as pl
from jax.experimental.pallas import tpu as pltpu
from jax.experimental.pallas import tpu_sc as plsc

N_EMBED = 1024
MAX_SEQ_LEN = 8192
BATCH = 4

SC_ROWS = 1024

S_BLK = 1024
TC_BLK0 = SC_ROWS // S_BLK
NUM_BLKS = (MAX_SEQ_LEN - SC_ROWS) // S_BLK


def _make_sc_broadcast():
    info = plsc.get_sparse_core_info()
    num_cores, num_subcores = info.num_cores, info.num_subcores
    num_workers = num_cores * num_subcores
    rows_per_worker = SC_ROWS // num_workers

    mesh = plsc.VectorSubcoreMesh(core_axis_name="c", subcore_axis_name="s")

    @functools.partial(
        pl.kernel,
        mesh=mesh,
        out_type=jax.ShapeDtypeStruct((BATCH, MAX_SEQ_LEN, N_EMBED), jnp.float32),
        scratch_types=[pltpu.VMEM((rows_per_worker, N_EMBED), jnp.float32)],
    )
    def broadcast_rows(table_hbm, out_hbm, buf):
        wid = lax.axis_index("s") * num_cores + lax.axis_index("c")
        base = wid * rows_per_worker
        pltpu.sync_copy(table_hbm.at[pl.ds(base, rows_per_worker)], buf)
        for b in range(BATCH):
            pltpu.sync_copy(buf, out_hbm.at[b, pl.ds(base, rows_per_worker)])

    return broadcast_rows


_sc_broadcast = _make_sc_broadcast()


def _tc_copy_body(table_ref, partial_ref, out_ref):
    del partial_ref
    blk = table_ref[...]
    for b in range(BATCH):
        out_ref[b] = blk


def _tc_fill_rest(pos_table, partial_out):
    return pl.pallas_call(
        _tc_copy_body,
        grid=(NUM_BLKS,),
        in_specs=[
            pl.BlockSpec((S_BLK, N_EMBED), lambda i: (i + TC_BLK0, 0)),
            pl.BlockSpec(memory_space=pl.ANY),
        ],
        out_specs=pl.BlockSpec((BATCH, S_BLK, N_EMBED), lambda i: (0, i + TC_BLK0, 0)),
        out_shape=jax.ShapeDtypeStruct((BATCH, MAX_SEQ_LEN, N_EMBED), jnp.float32),
        input_output_aliases={1: 0},
    )(pos_table, partial_out)


@jax.jit
def _broadcast(pos_table):
    partial_out = _sc_broadcast(pos_table)
    return _tc_fill_rest(pos_table, partial_out)


def kernel(input_ids, pos_table):
    del input_ids
    return _broadcast(pos_table)


# speedup vs baseline: 1.0077x; 1.0077x over previous
import functools

import jax
import jax.numpy as jnp
from jax import lax
from jax.experimental import pallas as pl
from jax.experimental.pallas import tpu as pltpu
from jax.experimental.pallas import tpu_sc as plsc

N_EMBED = 1024
MAX_SEQ_LEN = 8192
BATCH = 4

SC_ROWS = 2048

S_BLK = 1024
TC_BLK0 = SC_ROWS // S_BLK
NUM_BLKS = (MAX_SEQ_LEN - SC_ROWS) // S_BLK


def _make_sc_broadcast():
    info = plsc.get_sparse_core_info()
    num_cores, num_subcores = info.num_cores, info.num_subcores
    num_workers = num_cores * num_subcores
    rows_per_worker = SC_ROWS // num_workers

    mesh = plsc.VectorSubcoreMesh(core_axis_name="c", subcore_axis_name="s")

    @functools.partial(
        pl.kernel,
        mesh=mesh,
        out_type=jax.ShapeDtypeStruct((BATCH, MAX_SEQ_LEN, N_EMBED), jnp.float32),
        scratch_types=[pltpu.VMEM((rows_per_worker, N_EMBED), jnp.float32)],
    )
    def broadcast_rows(table_hbm, out_hbm, buf):
        wid = lax.axis_index("s") * num_cores + lax.axis_index("c")
        base = wid * rows_per_worker
        pltpu.sync_copy(table_hbm.at[pl.ds(base, rows_per_worker)], buf)
        for b in range(BATCH):
            pltpu.sync_copy(buf, out_hbm.at[b, pl.ds(base, rows_per_worker)])

    return broadcast_rows


_sc_broadcast = _make_sc_broadcast()


def _tc_copy_body(table_ref, partial_ref, out_ref):
    del partial_ref
    blk = table_ref[...]
    for b in range(BATCH):
        out_ref[b] = blk


def _tc_fill_rest(pos_table, partial_out):
    return pl.pallas_call(
        _tc_copy_body,
        grid=(NUM_BLKS,),
        in_specs=[
            pl.BlockSpec((S_BLK, N_EMBED), lambda i: (i + TC_BLK0, 0)),
            pl.BlockSpec(memory_space=pl.ANY),
        ],
        out_specs=pl.BlockSpec((BATCH, S_BLK, N_EMBED), lambda i: (0, i + TC_BLK0, 0)),
        out_shape=jax.ShapeDtypeStruct((BATCH, MAX_SEQ_LEN, N_EMBED), jnp.float32),
        input_output_aliases={1: 0},
    )(pos_table, partial_out)


@jax.jit
def _broadcast(pos_table):
    partial_out = _sc_broadcast(pos_table)
    return _tc_fill_rest(pos_table, partial_out)


def kernel(input_ids, pos_table):
    del input_ids
    return _broadcast(pos_table)
